# baseline (speedup 1.0000x reference)
import functools

import jax
import jax.numpy as jnp
from jax import lax
from jax.experimental import pallas as pl
from jax.experimental.pallas import tpu as pltpu
from jax.experimental.pallas import tpu_sc as plsc

DIM_CODES = 64
DICT_SIZE = 8192
DIM_EMBED = 32
BATCH = 128
K_CHUNK = 512


def _argmin_body(xt_ref, d_ref, idx_ref, kio_ref):
    @pl.when(pl.program_id(0) == 0)
    def _init_iota():
        kio_ref[...] = jax.lax.broadcasted_iota(
            jnp.int32, (DICT_SIZE, BATCH), 0)

    xt = xt_ref[0]
    dc = d_ref[0]
    xyT = jax.lax.dot_general(dc, xt, (((1,), (0,)), ((), ())),
                              preferred_element_type=jnp.float32)
    y_sq = jnp.sum(dc * dc, axis=1, keepdims=True)
    x_sq = jnp.sum(xt * xt, axis=0, keepdims=True)
    distT = x_sq - 2.0 * xyT + y_sq
    m = jnp.min(distT, axis=0, keepdims=True)
    kio = kio_ref[...]
    cand = jnp.where(distT == m, kio, DICT_SIZE)
    idx_ref[0] = jnp.min(cand, axis=0, keepdims=True)


def _onehot_body(idx_ref, out_ref):
    k0 = pl.program_id(0) * K_CHUNK
    kio = jax.lax.broadcasted_iota(jnp.int32, (BATCH, DIM_CODES, K_CHUNK), 2) + k0
    out_ref[...] = (kio == idx_ref[...][:, :, None]).astype(jnp.float32)


_NC = 2
_NW = 32
_ROWS = BATCH * DIM_CODES
_ROWS_PER_W = _ROWS // _NW


def _make_sc_gather():
    @functools.partial(
        pl.kernel,
        mesh=plsc.VectorSubcoreMesh(core_axis_name="c", subcore_axis_name="s"),
        out_type=jax.ShapeDtypeStruct((_ROWS, 4 * DIM_EMBED), jnp.float32),
        scratch_types=[
            pltpu.VMEM((_ROWS_PER_W,), jnp.int32),
            pltpu.VMEM((_ROWS_PER_W, 4 * DIM_EMBED), jnp.float32),
            pltpu.SemaphoreType.DMA,
        ],
    )
    def _sc_gather(table_hbm, idx_hbm, out_hbm, idx_v, rows_v, sem):
        wid = lax.axis_index("s") * _NC + lax.axis_index("c")
        base = wid * _ROWS_PER_W
        pltpu.sync_copy(idx_hbm.at[pl.ds(base, _ROWS_PER_W)], idx_v)
        pltpu.async_copy(table_hbm.at[idx_v], rows_v, sem).wait()
        pltpu.sync_copy(rows_v, out_hbm.at[pl.ds(base, _ROWS_PER_W)])

    return _sc_gather


def _extract_body(rows_ref, q_ref, out_ref):
    rows = rows_ref[...]
    q = q_ref[...]
    out = jnp.zeros((_ROWS, DIM_EMBED), jnp.float32)
    for j in range(4):
        out = jnp.where(q == j, rows[:, j * DIM_EMBED:(j + 1) * DIM_EMBED], out)
    out_ref[...] = out


def kernel(x, dictionary):
    xt = x.reshape(BATCH, DIM_CODES, DIM_EMBED).transpose(1, 2, 0)

    idx_t = pl.pallas_call(
        _argmin_body,
        grid=(DIM_CODES,),
        in_specs=[
            pl.BlockSpec((1, DIM_EMBED, BATCH), lambda c: (c, 0, 0)),
            pl.BlockSpec((1, DICT_SIZE, DIM_EMBED), lambda c: (c, 0, 0)),
        ],
        out_specs=pl.BlockSpec((1, 1, BATCH), lambda c: (c, 0, 0)),
        out_shape=jax.ShapeDtypeStruct((DIM_CODES, 1, BATCH), jnp.int32),
        scratch_shapes=[pltpu.VMEM((DICT_SIZE, BATCH), jnp.int32)],
    )(xt, dictionary)

    idx = idx_t.reshape(DIM_CODES, BATCH).transpose(1, 0)

    row_ids = (jnp.arange(DIM_CODES, dtype=jnp.int32)[None, :] * (DICT_SIZE // 4)
               + idx // 4).reshape(_ROWS)
    quad = (idx % 4).reshape(_ROWS, 1)
    table = dictionary.reshape(DIM_CODES * DICT_SIZE // 4, 4 * DIM_EMBED)
    rows4 = _make_sc_gather()(table, row_ids)
    cw_e = pl.pallas_call(
        _extract_body,
        in_specs=[
            pl.BlockSpec((_ROWS, 4 * DIM_EMBED), lambda: (0, 0)),
            pl.BlockSpec((_ROWS, 1), lambda: (0, 0)),
        ],
        out_specs=pl.BlockSpec((_ROWS, DIM_EMBED), lambda: (0, 0)),
        out_shape=jax.ShapeDtypeStruct((_ROWS, DIM_EMBED), jnp.float32),
    )(rows4, quad).reshape(BATCH, DIM_CODES * DIM_EMBED)

    one_hot = pl.pallas_call(
        _onehot_body,
        grid=(DICT_SIZE // K_CHUNK,),
        in_specs=[pl.BlockSpec((BATCH, DIM_CODES), lambda k: (0, 0))],
        out_specs=pl.BlockSpec((BATCH, DIM_CODES, K_CHUNK), lambda k: (0, 0, k)),
        out_shape=jax.ShapeDtypeStruct((BATCH, DIM_CODES, DICT_SIZE), jnp.float32),
    )(idx)

    return cw_e, cw_e, one_hot


# speedup vs baseline: 1.4014x; 1.4014x over previous
import jax
import jax.numpy as jnp
from jax.experimental import pallas as pl
from jax.experimental.pallas import tpu as pltpu

DIM_CODES = 64
DICT_SIZE = 8192
DIM_EMBED = 32
BATCH = 128
K_CHUNK = 512


def _argmin_body(xt_ref, d_ref, idx_ref, ce_ref, kio_ref):
    @pl.when(pl.program_id(0) == 0)
    def _init_iota():
        kio_ref[...] = jax.lax.broadcasted_iota(
            jnp.int32, (DICT_SIZE, BATCH), 0)

    xt = xt_ref[0]
    dc = d_ref[0]
    xyT = jax.lax.dot_general(dc, xt, (((1,), (0,)), ((), ())),
                              preferred_element_type=jnp.float32)
    y_sq = jnp.sum(dc * dc, axis=1, keepdims=True)
    x_sq = jnp.sum(xt * xt, axis=0, keepdims=True)
    distT = x_sq - 2.0 * xyT + y_sq
    m = jnp.min(distT, axis=0, keepdims=True)
    kio = kio_ref[...]
    cand = jnp.where(distT == m, kio, DICT_SIZE)
    idxv = jnp.min(cand, axis=0, keepdims=True)
    idx_ref[0] = idxv
    onehotT = (kio == idxv).astype(jnp.float32)
    ceT = jax.lax.dot_general(dc, onehotT, (((0,), (0,)), ((), ())),
                              preferred_element_type=jnp.float32)
    ce_ref[0] = ceT


def _onehot_body(idx_ref, out_ref):
    k0 = pl.program_id(0) * K_CHUNK
    kio = jax.lax.broadcasted_iota(jnp.int32, (BATCH, DIM_CODES, K_CHUNK), 2) + k0
    out_ref[...] = (kio == idx_ref[...][:, :, None]).astype(jnp.float32)


def kernel(x, dictionary):
    xt = x.reshape(BATCH, DIM_CODES, DIM_EMBED).transpose(1, 2, 0)

    idx_t, ce_t = pl.pallas_call(
        _argmin_body,
        grid=(DIM_CODES,),
        in_specs=[
            pl.BlockSpec((1, DIM_EMBED, BATCH), lambda c: (c, 0, 0)),
            pl.BlockSpec((1, DICT_SIZE, DIM_EMBED), lambda c: (c, 0, 0)),
        ],
        out_specs=[
            pl.BlockSpec((1, 1, BATCH), lambda c: (c, 0, 0)),
            pl.BlockSpec((1, DIM_EMBED, BATCH), lambda c: (c, 0, 0)),
        ],
        out_shape=[
            jax.ShapeDtypeStruct((DIM_CODES, 1, BATCH), jnp.int32),
            jax.ShapeDtypeStruct((DIM_CODES, DIM_EMBED, BATCH), jnp.float32),
        ],
        scratch_shapes=[pltpu.VMEM((DICT_SIZE, BATCH), jnp.int32)],
    )(xt, dictionary)

    idx = idx_t.reshape(DIM_CODES, BATCH).transpose(1, 0)
    cw_e = ce_t.transpose(2, 0, 1).reshape(BATCH, DIM_CODES * DIM_EMBED)

    one_hot = pl.pallas_call(
        _onehot_body,
        grid=(DICT_SIZE // K_CHUNK,),
        in_specs=[pl.BlockSpec((BATCH, DIM_CODES), lambda k: (0, 0))],
        out_specs=pl.BlockSpec((BATCH, DIM_CODES, K_CHUNK), lambda k: (0, 0, k)),
        out_shape=jax.ShapeDtypeStruct((BATCH, DIM_CODES, DICT_SIZE), jnp.float32),
    )(idx)

    return cw_e, cw_e, one_hot
